# baseline (speedup 1.0000x reference)
import functools

import jax
import jax.numpy as jnp
from jax import lax
from jax.experimental import pallas as pl
from jax.experimental.pallas import tpu as pltpu
from jax.experimental.pallas import tpu_sc as plsc

_WIN = 4
_RING = 16


def _sc_gather(tableT, idx):
    D, V = tableT.shape
    B = idx.shape[0]
    info = plsc.get_sparse_core_info()
    num_workers = info.num_cores * info.num_subcores
    n_blocks = (V + 127) // 128
    blk_per_w = (n_blocks + num_workers - 1) // num_workers
    blk_pad = 256
    n_win = blk_pad // _WIN
    qcap = 528
    cap = 1024
    n_groups = B // 16
    mesh = plsc.VectorSubcoreMesh(core_axis_name="c", subcore_axis_name="s")

    @functools.partial(
        pl.kernel,
        mesh=mesh,
        out_type=jax.ShapeDtypeStruct((B, D), jnp.float32),
        scratch_types=[
            pltpu.VMEM((B,), jnp.int32),
            pltpu.VMEM((cap + 16,), jnp.int32),
            pltpu.VMEM((cap + 16,), jnp.int32),
            pltpu.VMEM((4 * 528 + 16,), jnp.int32),
            pltpu.VMEM((4 * 528 + 16,), jnp.int32),
            pltpu.VMEM((16,), jnp.int32),
            pltpu.VMEM((16,), jnp.int32),
            pltpu.VMEM((16,), jnp.int32),
            pltpu.VMEM((16,), jnp.int32),
            pltpu.VMEM((_RING, D), jnp.float32),
            pltpu.VMEM((2 * _WIN, D, 128), jnp.float32),
            [pltpu.SemaphoreType.DMA for _ in range(2)],
            pltpu.SemaphoreType.DMA,
        ],
        compiler_params=pltpu.CompilerParams(
            use_tc_tiling_on_sc=True, needs_layout_passes=False
        ),
    )
    def gather_kernel(
        table_hbm,
        idx_hbm,
        out_hbm,
        idx_all,
        my_idx,
        my_pos,
        q_idx,
        q_pos,
        qcnt,
        tmpi,
        tmpp,
        tmpr,
        ring,
        blocks,
        sems,
        osem,
    ):
        wid = lax.axis_index("s") * info.num_cores + lax.axis_index("c")
        lo = wid * blk_per_w
        hi = lo + blk_per_w
        pltpu.sync_copy(idx_hbm, idx_all)
        lane = lax.iota(jnp.int32, 16)

        def compact(g, n):
            v = idx_all[pl.ds(g * 16, 16)]
            cb = v >> 7
            mine = (cb >= lo) & (cb < hi)
            plsc.store_compressed(my_idx.at[pl.ds(n, 16)], v, mask=mine)
            plsc.store_compressed(
                my_pos.at[pl.ds(n, 16)], lane + g * 16, mask=mine
            )
            return n + plsc.all_reduce_population_count(mine)[0]

        my_n = lax.fori_loop(0, n_groups, compact, jnp.int32(0))
        my_g = (my_n + 15) >> 4

        def bucket(t, ns):
            u = my_idx[pl.ds(t * 16, 16)]
            pj = my_pos[pl.ds(t * 16, 16)]
            valid = (lane + t * 16) < my_n
            qq = ((u >> 7) - lo) >> 6
            ns2 = []
            for qs in range(4):
                m = (qq == qs) & valid
                plsc.store_compressed(
                    q_idx.at[pl.ds(qs * qcap + ns[qs], 16)], u, mask=m
                )
                plsc.store_compressed(
                    q_pos.at[pl.ds(qs * qcap + ns[qs], 16)], pj, mask=m
                )
                ns2.append(ns[qs] + plsc.all_reduce_population_count(m)[0])
            return tuple(ns2)

        nq = lax.fori_loop(
            0, my_g, bucket, (jnp.int32(0),) * 4
        )
        qv = jnp.full((16,), nq[0])
        for qs in range(1, 4):
            qv = jnp.where(lane == qs, nq[qs], qv)
        qcnt[pl.ds(0, 16)] = qv

        def fetch_window(w, par):
            for s in range(_WIN):
                b = jnp.minimum(lo + w * _WIN + s, n_blocks - 1)
                off = pl.multiple_of(b * 128, 128)
                pltpu.async_copy(
                    table_hbm.at[:, pl.ds(off, 128)],
                    blocks.at[par * _WIN + s],
                    sems[par],
                )

        def wait_window(par):
            for s in range(_WIN):
                pltpu.make_async_copy(
                    table_hbm.at[:, pl.ds(0, 128)],
                    blocks.at[par * _WIN + s],
                    sems[par],
                ).wait()

        fetch_window(jnp.int32(0), 0)

        def step(w, rc):
            par = (w % 2).astype(jnp.int32)
            for par_s in range(2):

                @pl.when(par == par_s)
                def _():
                    wait_window(par_s)

                    @pl.when(w + 1 < n_win)
                    def _():
                        fetch_window(w + 1, 1 - par_s)

            wbase = lo + w * _WIN
            wq = w >> 4
            qbase = wq * qcap
            nq_w = jnp.take(qcnt[pl.ds(0, 16)], jnp.full((16,), wq))[0]
            ng_w = (nq_w + 15) >> 4

            def scan(t, rc):
                u = q_idx[pl.ds(qbase + t * 16, 16)]
                valid = (lane + t * 16) < nq_w
                rel = (u >> 7) - wbase
                match = (rel >= 0) & (rel < _WIN) & valid
                nm = plsc.all_reduce_population_count(match)[0]

                def emit(k, rc):
                    plsc.store_compressed(
                        tmpi.at[pl.ds(0, 16)], u, mask=match
                    )
                    plsc.store_compressed(
                        tmpr.at[pl.ds(0, 16)], rel + par * _WIN, mask=match
                    )
                    pj = q_pos[pl.ds(qbase + t * 16, 16)]
                    plsc.store_compressed(
                        tmpp.at[pl.ds(0, 16)], pj, mask=match
                    )
                    k16 = jnp.full((16,), k)
                    l16 = jnp.take(tmpi[pl.ds(0, 16)], k16) & 127
                    s16 = jnp.take(tmpr[pl.ds(0, 16)], k16)
                    r = rc % _RING

                    @pl.when(rc >= _RING)
                    def _():
                        pltpu.make_async_copy(
                            out_hbm.at[0], ring.at[0], osem
                        ).wait()

                    for q in range(D // 16):
                        xg = plsc.load_gather(
                            blocks, [s16, lane + q * 16, l16]
                        )
                        ring[r, pl.ds(q * 16, 16)] = xg
                    p0 = jnp.take(tmpp[pl.ds(0, 16)], k16)[0]
                    pltpu.async_copy(ring.at[r], out_hbm.at[p0], osem)
                    return rc + 1

                return lax.fori_loop(0, nm, emit, rc)

            return lax.fori_loop(0, ng_w, scan, rc)

        rc = lax.fori_loop(0, n_win, step, jnp.int32(0))

        def drain(j, _):
            pltpu.make_async_copy(
                out_hbm.at[0], ring.at[0], osem
            ).wait()
            return _

        lax.fori_loop(0, jnp.minimum(rc, _RING), drain, None)

    return gather_kernel(tableT, idx)


def _tc_mlp(x, W1, b1, W2T, b2):
    B, D = x.shape
    H = W1.shape[1]
    BLK = 2048
    dn = (((1,), (1,)), ((), ()))

    def body(x_ref, w1_ref, b1_ref, w2t_ref, b2_ref, o_ref):
        xb = x_ref[...]
        h = jnp.maximum(
            jnp.dot(xb, w1_ref[...], preferred_element_type=jnp.float32)
            + b1_ref[...],
            0.0,
        )
        y = jnp.maximum(
            xb
            + lax.dot_general(
                h, w2t_ref[...], dn, preferred_element_type=jnp.float32
            )
            + b2_ref[...],
            0.0,
        )
        o_ref[...] = y.T

    return pl.pallas_call(
        body,
        grid=(B // BLK,),
        in_specs=[
            pl.BlockSpec((BLK, D), lambda i: (i, 0)),
            pl.BlockSpec((D, H), lambda i: (0, 0)),
            pl.BlockSpec((1, H), lambda i: (0, 0)),
            pl.BlockSpec((D, H), lambda i: (0, 0)),
            pl.BlockSpec((1, D), lambda i: (0, 0)),
        ],
        out_specs=pl.BlockSpec((D, BLK), lambda i: (0, i)),
        out_shape=jax.ShapeDtypeStruct((D, B), jnp.float32),
    )(x, W1, b1.reshape(1, H), W2T, b2.reshape(1, D))


def kernel(a, table, W1, b1, W2, b2):
    x = _sc_gather(table.T, a.astype(jnp.int32))
    outT = _tc_mlp(x, W1, b1, W2.T, b2)
    return outT.T


# speedup vs baseline: 1.0258x; 1.0258x over previous
import functools

import jax
import jax.numpy as jnp
from jax import lax
from jax.experimental import pallas as pl
from jax.experimental.pallas import tpu as pltpu
from jax.experimental.pallas import tpu_sc as plsc

_WIN = 4
_RING = 16


def _sc_gather(tableT, idx):
    D, V = tableT.shape
    B = idx.shape[0]
    info = plsc.get_sparse_core_info()
    num_workers = info.num_cores * info.num_subcores
    n_blocks = (V + 127) // 128
    blk_per_w = (n_blocks + num_workers - 1) // num_workers
    n_win = (blk_per_w + _WIN - 1) // _WIN
    cap = 1024
    n_groups = B // 16
    mesh = plsc.VectorSubcoreMesh(core_axis_name="c", subcore_axis_name="s")

    @functools.partial(
        pl.kernel,
        mesh=mesh,
        out_type=jax.ShapeDtypeStruct((B, D), jnp.float32),
        scratch_types=[
            pltpu.VMEM((B,), jnp.int32),
            pltpu.VMEM((cap + 16,), jnp.int32),
            pltpu.VMEM((cap + 16,), jnp.int32),
            pltpu.VMEM((16,), jnp.int32),
            pltpu.VMEM((16,), jnp.int32),
            pltpu.VMEM((16,), jnp.int32),
            pltpu.VMEM((_RING, D), jnp.float32),
            pltpu.VMEM((2 * _WIN, D, 128), jnp.float32),
            [pltpu.SemaphoreType.DMA for _ in range(2)],
            pltpu.SemaphoreType.DMA,
        ],
        compiler_params=pltpu.CompilerParams(
            use_tc_tiling_on_sc=True, needs_layout_passes=False
        ),
    )
    def gather_kernel(
        table_hbm,
        idx_hbm,
        out_hbm,
        idx_all,
        my_idx,
        my_pos,
        tmpi,
        tmpp,
        tmpr,
        ring,
        blocks,
        sems,
        osem,
    ):
        wid = lax.axis_index("s") * info.num_cores + lax.axis_index("c")
        lo = wid * blk_per_w
        hi = lo + blk_per_w
        pltpu.sync_copy(idx_hbm, idx_all)
        lane = lax.iota(jnp.int32, 16)

        def compact(g, n):
            v = idx_all[pl.ds(g * 16, 16)]
            cb = v >> 7
            mine = (cb >= lo) & (cb < hi)
            plsc.store_compressed(my_idx.at[pl.ds(n, 16)], v, mask=mine)
            plsc.store_compressed(
                my_pos.at[pl.ds(n, 16)], lane + g * 16, mask=mine
            )
            return n + plsc.all_reduce_population_count(mine)[0]

        my_n = lax.fori_loop(0, n_groups, compact, jnp.int32(0))
        my_g = (my_n + 15) >> 4

        def fetch_window(w, par):
            for s in range(_WIN):
                b = jnp.minimum(lo + w * _WIN + s, n_blocks - 1)
                off = pl.multiple_of(b * 128, 128)
                pltpu.async_copy(
                    table_hbm.at[:, pl.ds(off, 128)],
                    blocks.at[par * _WIN + s],
                    sems[par],
                )

        def wait_window(par):
            for s in range(_WIN):
                pltpu.make_async_copy(
                    table_hbm.at[:, pl.ds(0, 128)],
                    blocks.at[par * _WIN + s],
                    sems[par],
                ).wait()

        fetch_window(jnp.int32(0), 0)

        def step(w, rc):
            par = (w % 2).astype(jnp.int32)
            for par_s in range(2):

                @pl.when(par == par_s)
                def _():
                    wait_window(par_s)

                    @pl.when(w + 1 < n_win)
                    def _():
                        fetch_window(w + 1, 1 - par_s)

            wbase = lo + w * _WIN

            def scan(t, rc):
                u = my_idx[pl.ds(t * 16, 16)]
                valid = (lane + t * 16) < my_n
                rel = (u >> 7) - wbase
                match = (rel >= 0) & (rel < _WIN) & valid
                nm = plsc.all_reduce_population_count(match)[0]

                def emit(k, rc):
                    plsc.store_compressed(
                        tmpi.at[pl.ds(0, 16)], u, mask=match
                    )
                    plsc.store_compressed(
                        tmpr.at[pl.ds(0, 16)], rel + par * _WIN, mask=match
                    )
                    pj = my_pos[pl.ds(t * 16, 16)]
                    plsc.store_compressed(
                        tmpp.at[pl.ds(0, 16)], pj, mask=match
                    )
                    k16 = jnp.full((16,), k)
                    l16 = jnp.take(tmpi[pl.ds(0, 16)], k16) & 127
                    s16 = jnp.take(tmpr[pl.ds(0, 16)], k16)
                    r = rc % _RING

                    @pl.when(rc >= _RING)
                    def _():
                        pltpu.make_async_copy(
                            out_hbm.at[0], ring.at[0], osem
                        ).wait()

                    for q in range(D // 16):
                        xg = plsc.load_gather(
                            blocks, [s16, lane + q * 16, l16]
                        )
                        ring[r, pl.ds(q * 16, 16)] = xg
                    p0 = jnp.take(tmpp[pl.ds(0, 16)], k16)[0]
                    pltpu.async_copy(ring.at[r], out_hbm.at[p0], osem)
                    return rc + 1

                return lax.fori_loop(0, nm, emit, rc)

            return lax.fori_loop(0, my_g, scan, rc)

        rc = lax.fori_loop(0, n_win, step, jnp.int32(0))

        def drain(j, _):
            pltpu.make_async_copy(
                out_hbm.at[0], ring.at[0], osem
            ).wait()
            return _

        lax.fori_loop(0, jnp.minimum(rc, _RING), drain, None)

    return gather_kernel(tableT, idx)


def _tc_mlp(x, W1, b1, W2T, b2):
    B, D = x.shape
    H = W1.shape[1]
    BLK = 2048
    dn = (((1,), (1,)), ((), ()))

    def body(x_ref, w1_ref, b1_ref, w2t_ref, b2_ref, o_ref):
        xb = x_ref[...]
        h = jnp.maximum(
            jnp.dot(xb, w1_ref[...], preferred_element_type=jnp.float32)
            + b1_ref[...],
            0.0,
        )
        y = jnp.maximum(
            xb
            + lax.dot_general(
                h, w2t_ref[...], dn, preferred_element_type=jnp.float32
            )
            + b2_ref[...],
            0.0,
        )
        o_ref[...] = y.T

    return pl.pallas_call(
        body,
        grid=(B // BLK,),
        in_specs=[
            pl.BlockSpec((BLK, D), lambda i: (i, 0)),
            pl.BlockSpec((D, H), lambda i: (0, 0)),
            pl.BlockSpec((1, H), lambda i: (0, 0)),
            pl.BlockSpec((D, H), lambda i: (0, 0)),
            pl.BlockSpec((1, D), lambda i: (0, 0)),
        ],
        out_specs=pl.BlockSpec((D, BLK), lambda i: (0, i)),
        out_shape=jax.ShapeDtypeStruct((D, B), jnp.float32),
    )(x, W1, b1.reshape(1, H), W2T, b2.reshape(1, D))


def kernel(a, table, W1, b1, W2, b2):
    x = _sc_gather(table.T, a.astype(jnp.int32))
    outT = _tc_mlp(x, W1, b1, W2.T, b2)
    return outT.T


# speedup vs baseline: 1.0356x; 1.0095x over previous
import functools

import jax
import jax.numpy as jnp
from jax import lax
from jax.experimental import pallas as pl
from jax.experimental.pallas import tpu as pltpu
from jax.experimental.pallas import tpu_sc as plsc

_WIN = 4
_RING = 16


def _sc_gather(tableT, idx):
    D, V = tableT.shape
    B = idx.shape[0]
    info = plsc.get_sparse_core_info()
    num_workers = info.num_cores * info.num_subcores
    n_blocks = (V + 127) // 128
    blk_per_w = (n_blocks + num_workers - 1) // num_workers
    n_win = (blk_per_w + _WIN - 1) // _WIN
    cap = 1024
    n_groups = B // 16
    mesh = plsc.VectorSubcoreMesh(core_axis_name="c", subcore_axis_name="s")

    @functools.partial(
        pl.kernel,
        mesh=mesh,
        out_type=jax.ShapeDtypeStruct((B, D), jnp.float32),
        scratch_types=[
            pltpu.VMEM((B,), jnp.int32),
            pltpu.VMEM((cap + 16,), jnp.int32),
            pltpu.VMEM((cap + 16,), jnp.int32),
            pltpu.VMEM((16,), jnp.int32),
            pltpu.VMEM((16,), jnp.int32),
            pltpu.VMEM((16,), jnp.int32),
            pltpu.VMEM((_RING, D), jnp.float32),
            pltpu.VMEM((2 * _WIN, D, 128), jnp.float32),
            [pltpu.SemaphoreType.DMA for _ in range(2)],
            pltpu.SemaphoreType.DMA,
        ],
        compiler_params=pltpu.CompilerParams(
            use_tc_tiling_on_sc=True, needs_layout_passes=False
        ),
    )
    def gather_kernel(
        table_hbm,
        idx_hbm,
        out_hbm,
        idx_all,
        my_idx,
        my_pos,
        tmpi,
        tmpp,
        tmpr,
        ring,
        blocks,
        sems,
        osem,
    ):
        wid = lax.axis_index("s") * info.num_cores + lax.axis_index("c")
        lo = wid * blk_per_w
        hi = lo + blk_per_w
        pltpu.sync_copy(idx_hbm, idx_all)
        lane = lax.iota(jnp.int32, 16)

        def compact(g, n):
            v = idx_all[pl.ds(g * 16, 16)]
            cb = v >> 7
            mine = (cb >= lo) & (cb < hi)
            plsc.store_compressed(my_idx.at[pl.ds(n, 16)], v, mask=mine)
            plsc.store_compressed(
                my_pos.at[pl.ds(n, 16)], lane + g * 16, mask=mine
            )
            n = n + plsc.all_reduce_population_count(mine)[0]
            return jnp.minimum(n, cap)

        my_n = lax.fori_loop(0, n_groups, compact, jnp.int32(0))
        my_g = (my_n + 15) >> 4

        def fetch_window(w, par):
            for s in range(_WIN):
                b = jnp.minimum(lo + w * _WIN + s, n_blocks - 1)
                off = pl.multiple_of(b * 128, 128)
                pltpu.async_copy(
                    table_hbm.at[:, pl.ds(off, 128)],
                    blocks.at[par * _WIN + s],
                    sems[par],
                )

        def wait_window(par):
            for s in range(_WIN):
                pltpu.make_async_copy(
                    table_hbm.at[:, pl.ds(0, 128)],
                    blocks.at[par * _WIN + s],
                    sems[par],
                ).wait()

        fetch_window(jnp.int32(0), 0)

        def step(w, rc):
            par = (w % 2).astype(jnp.int32)
            for par_s in range(2):

                @pl.when(par == par_s)
                def _():
                    wait_window(par_s)

                    @pl.when(w + 1 < n_win)
                    def _():
                        fetch_window(w + 1, 1 - par_s)

            wbase = lo + w * _WIN

            def scan(t, rc):
                u = my_idx[pl.ds(t * 16, 16)]
                valid = (lane + t * 16) < my_n
                rel = (u >> 7) - wbase
                match = (rel >= 0) & (rel < _WIN) & valid
                nm = plsc.all_reduce_population_count(match)[0]

                def emit(k, rc):
                    plsc.store_compressed(
                        tmpi.at[pl.ds(0, 16)], u, mask=match
                    )
                    plsc.store_compressed(
                        tmpr.at[pl.ds(0, 16)], rel + par * _WIN, mask=match
                    )
                    pj = my_pos[pl.ds(t * 16, 16)]
                    plsc.store_compressed(
                        tmpp.at[pl.ds(0, 16)], pj, mask=match
                    )
                    k16 = jnp.full((16,), k)
                    l16 = jnp.take(tmpi[pl.ds(0, 16)], k16) & 127
                    s16 = jnp.take(tmpr[pl.ds(0, 16)], k16)
                    r = rc % _RING

                    @pl.when(rc >= _RING)
                    def _():
                        pltpu.make_async_copy(
                            out_hbm.at[0], ring.at[0], osem
                        ).wait()

                    for q in range(D // 16):
                        xg = plsc.load_gather(
                            blocks, [s16, lane + q * 16, l16]
                        )
                        ring[r, pl.ds(q * 16, 16)] = xg
                    p0 = jnp.take(tmpp[pl.ds(0, 16)], k16)[0]
                    pltpu.async_copy(ring.at[r], out_hbm.at[p0], osem)
                    return rc + 1

                return lax.fori_loop(0, nm, emit, rc)

            return lax.fori_loop(0, my_g, scan, rc)

        rc = lax.fori_loop(0, n_win, step, jnp.int32(0))

        def drain(j, _):
            pltpu.make_async_copy(
                out_hbm.at[0], ring.at[0], osem
            ).wait()
            return _

        lax.fori_loop(0, jnp.minimum(rc, _RING), drain, None)

    return gather_kernel(tableT, idx)


def _tc_mlp(x, W1, b1, W2T, b2):
    B, D = x.shape
    H = W1.shape[1]
    BLK = 2048
    dn = (((1,), (1,)), ((), ()))

    def body(x_ref, w1_ref, b1_ref, w2t_ref, b2_ref, o_ref):
        xb = x_ref[...]
        h = jnp.maximum(
            jnp.dot(xb, w1_ref[...], preferred_element_type=jnp.float32)
            + b1_ref[...],
            0.0,
        )
        y = jnp.maximum(
            xb
            + lax.dot_general(
                h, w2t_ref[...], dn, preferred_element_type=jnp.float32
            )
            + b2_ref[...],
            0.0,
        )
        o_ref[...] = y.T

    return pl.pallas_call(
        body,
        grid=(B // BLK,),
        in_specs=[
            pl.BlockSpec((BLK, D), lambda i: (i, 0)),
            pl.BlockSpec((D, H), lambda i: (0, 0)),
            pl.BlockSpec((1, H), lambda i: (0, 0)),
            pl.BlockSpec((D, H), lambda i: (0, 0)),
            pl.BlockSpec((1, D), lambda i: (0, 0)),
        ],
        out_specs=pl.BlockSpec((D, BLK), lambda i: (0, i)),
        out_shape=jax.ShapeDtypeStruct((D, B), jnp.float32),
    )(x, W1, b1.reshape(1, H), W2T, b2.reshape(1, D))


def kernel(a, table, W1, b1, W2, b2):
    x = _sc_gather(table.T, a.astype(jnp.int32))
    outT = _tc_mlp(x, W1, b1, W2.T, b2)
    return outT.T
